# baseline (speedup 1.0000x reference)
import functools

import jax
import jax.numpy as jnp
from jax import lax
from jax.experimental import pallas as pl
from jax.experimental.pallas import tpu as pltpu
from jax.experimental.pallas import tpu_sc as plsc

B = 4096
S = 8192
LANES = 16
NUM_CORES = 2
NUM_SUBCORES = 16
NUM_WORKERS = NUM_CORES * NUM_SUBCORES
ROWS_PER_WORKER = B // NUM_WORKERS
RG = LANES
NGROUPS = ROWS_PER_WORKER // RG
SEG = 1024
STRIDE = SEG + 1
CCHUNKS = S // SEG
NTILES = NGROUPS * CCHUNKS
NBUF = 4
TGROUPS = NTILES // NBUF
UG = 8

_mesh = plsc.VectorSubcoreMesh(core_axis_name="c", subcore_axis_name="s")


@functools.partial(
    pl.kernel,
    mesh=_mesh,
    out_type=jax.ShapeDtypeStruct((B, S), jnp.float32),
    scratch_types=(
        [pltpu.VMEM((RG, STRIDE), jnp.float32)] * NBUF
        + [pltpu.SemaphoreType.DMA] * (2 * NBUF)
    ),
    compiler_params=pltpu.CompilerParams(needs_layout_passes=False),
)
def _cumsum_sc(x_hbm, out_hbm, *scratch):
    bufs = scratch[:NBUF]
    lsems = scratch[NBUF:2 * NBUF]
    ssems = scratch[2 * NBUF:]

    wid = lax.axis_index("s") * NUM_CORES + lax.axis_index("c")
    base_row = wid * ROWS_PER_WORKER
    iota = lax.broadcasted_iota(jnp.int32, (LANES,), 0)

    def hbm_slice(t):
        rg = lax.div(t, CCHUNKS)
        cc = lax.rem(t, CCHUNKS)
        return (pl.ds(base_row + rg * RG, RG), pl.ds(cc * SEG, SEG))

    def start_load(t, b):
        r, c = hbm_slice(t)
        pltpu.make_async_copy(
            x_hbm.at[r, c], bufs[b].at[:, pl.ds(0, SEG)], lsems[b]).start()

    def wait_load(b):
        pltpu.make_async_copy(
            x_hbm.at[pl.ds(base_row, RG), pl.ds(0, SEG)],
            bufs[b].at[:, pl.ds(0, SEG)], lsems[b]).wait()

    def start_store(t, b):
        r, c = hbm_slice(t)
        pltpu.make_async_copy(
            bufs[b].at[:, pl.ds(0, SEG)], out_hbm.at[r, c], ssems[b]).start()

    def wait_store(b):
        pltpu.make_async_copy(
            bufs[b].at[:, pl.ds(0, SEG)],
            out_hbm.at[pl.ds(base_row, RG), pl.ds(0, SEG)], ssems[b]).wait()

    def compute_tile(b, acc):
        col0 = jnp.zeros((LANES,), jnp.int32)

        @plsc.parallel_loop(0, SEG // UG, carry=(acc, col0))
        def body(jj, carry):
            acc, col = carry
            for u in range(UG):
                idx_c = col + u
                v = plsc.load_gather(bufs[b], [iota, idx_c])
                acc = acc + v
                plsc.store_scatter(bufs[b], [iota, idx_c], acc)
            return acc, col + UG

        return body[0]

    start_load(0, 0)

    def group_body(g, acc):
        for u in range(NBUF):
            t = g * NBUF + u
            bn = (u + 1) % NBUF
            nc = t + 1

            @pl.when(nc < NTILES)
            def _prefetch():
                @pl.when(nc >= NBUF)
                def _drain():
                    wait_store(bn)
                start_load(nc, bn)

            wait_load(u)
            if u == 0:
                keep = jnp.where(lax.rem(t, CCHUNKS) == 0,
                                 jnp.float32(0.0), jnp.float32(1.0))
                acc = acc * keep
            acc = compute_tile(u, acc)
            start_store(t, u)
        return acc

    lax.fori_loop(0, TGROUPS, group_body, jnp.zeros((LANES,), jnp.float32))

    for b in range(NBUF):
        wait_store(b)


def kernel(x):
    return _cumsum_sc(x)


# speedup vs baseline: 3.3797x; 3.3797x over previous
import functools

import jax
import jax.numpy as jnp
from jax import lax
from jax.experimental import pallas as pl
from jax.experimental.pallas import tpu as pltpu
from jax.experimental.pallas import tpu_sc as plsc

B = 4096
S = 8192
LANES = 16
NUM_CORES = 2
NUM_SUBCORES = 16
NUM_WORKERS = NUM_CORES * NUM_SUBCORES
ROWS_PER_WORKER = B // NUM_WORKERS
RG = LANES
NGROUPS = ROWS_PER_WORKER // RG
SEG = 1024
STRIDE = SEG + 1
CCHUNKS = S // SEG
NTILES = NGROUPS * CCHUNKS
NBUF = 4
TGROUPS = NTILES // NBUF
UG = 8

_mesh = plsc.VectorSubcoreMesh(core_axis_name="c", subcore_axis_name="s")


@functools.partial(
    pl.kernel,
    mesh=_mesh,
    out_type=jax.ShapeDtypeStruct((B, S), jnp.float32),
    scratch_types=(
        [pltpu.VMEM((RG, STRIDE), jnp.float32)] * NBUF
        + [pltpu.SemaphoreType.DMA] * (2 * NBUF)
    ),
    compiler_params=pltpu.CompilerParams(
        needs_layout_passes=False, use_tc_tiling_on_sc=False),
)
def _cumsum_sc(x_hbm, out_hbm, *scratch):
    bufs = scratch[:NBUF]
    lsems = scratch[NBUF:2 * NBUF]
    ssems = scratch[2 * NBUF:]

    wid = lax.axis_index("s") * NUM_CORES + lax.axis_index("c")
    base_row = wid * ROWS_PER_WORKER
    iota = lax.broadcasted_iota(jnp.int32, (LANES,), 0)

    def hbm_slice(t):
        rg = lax.div(t, CCHUNKS)
        cc = lax.rem(t, CCHUNKS)
        return (pl.ds(base_row + rg * RG, RG), pl.ds(cc * SEG, SEG))

    def start_load(t, b):
        r, c = hbm_slice(t)
        pltpu.make_async_copy(
            x_hbm.at[r, c], bufs[b].at[:, pl.ds(0, SEG)], lsems[b]).start()

    def wait_load(b):
        pltpu.make_async_copy(
            x_hbm.at[pl.ds(base_row, RG), pl.ds(0, SEG)],
            bufs[b].at[:, pl.ds(0, SEG)], lsems[b]).wait()

    def start_store(t, b):
        r, c = hbm_slice(t)
        pltpu.make_async_copy(
            bufs[b].at[:, pl.ds(0, SEG)], out_hbm.at[r, c], ssems[b]).start()

    def wait_store(b):
        pltpu.make_async_copy(
            bufs[b].at[:, pl.ds(0, SEG)],
            out_hbm.at[pl.ds(base_row, RG), pl.ds(0, SEG)], ssems[b]).wait()

    def compute_tile(b, acc):
        col0 = jnp.zeros((LANES,), jnp.int32)

        @plsc.parallel_loop(0, SEG // UG, carry=(acc, col0))
        def body(jj, carry):
            acc, col = carry
            idxs = [col + u for u in range(UG)]
            vals = [plsc.load_gather(bufs[b], [iota, ix]) for ix in idxs]
            outs = []
            for u in range(UG):
                acc = acc + vals[u]
                outs.append(acc)
            for u in range(UG):
                plsc.store_scatter(bufs[b], [iota, idxs[u]], outs[u])
            return acc, col + UG

        return body[0]

    start_load(0, 0)

    def group_body(g, acc):
        for u in range(NBUF):
            t = g * NBUF + u
            bn = (u + 1) % NBUF
            nc = t + 1

            @pl.when(nc < NTILES)
            def _prefetch():
                @pl.when(nc >= NBUF)
                def _drain():
                    wait_store(bn)
                start_load(nc, bn)

            wait_load(u)
            if u == 0:
                keep = jnp.where(lax.rem(t, CCHUNKS) == 0,
                                 jnp.float32(0.0), jnp.float32(1.0))
                acc = acc * keep
            acc = compute_tile(u, acc)
            start_store(t, u)
        return acc

    lax.fori_loop(0, TGROUPS, group_body, jnp.zeros((LANES,), jnp.float32))

    for b in range(NBUF):
        wait_store(b)


def kernel(x):
    return _cumsum_sc(x)
